# Initial kernel scaffold; baseline (speedup 1.0000x reference)
import jax
import jax.numpy as jnp
from jax.experimental import pallas as pl


def kernel(x, Wg, w1, w3, w2):
    raise NotImplementedError("write your pallas kernel here")


# speedup vs baseline: 1.0903x; 1.0903x over previous
import functools

import jax
import jax.numpy as jnp
from jax.experimental import pallas as pl
from jax.experimental.pallas import tpu as pltpu

NUM_EXPERTS = 8
TOP_K = 2
D_MODEL = 1024
D_FF = 2048

TM = 512


def _moe_body(wvec_ref, x_ref, w1_ref, w3_ref, w2_ref, o_ref):
    e = pl.program_id(1)
    xb = x_ref[...]
    h1 = jnp.dot(xb, w1_ref[0], preferred_element_type=jnp.float32)
    h3 = jnp.dot(xb, w3_ref[0], preferred_element_type=jnp.float32)
    h = (h1 * jax.nn.sigmoid(h1)) * h3
    y = jnp.dot(h, w2_ref[0], preferred_element_type=jnp.float32)
    contrib = wvec_ref[0, 0, :][:, None] * y

    @pl.when(e == 0)
    def _init():
        o_ref[...] = contrib

    @pl.when(e != 0)
    def _acc():
        o_ref[...] += contrib


def kernel(x, Wg, w1, w3, w2):
    B, S, D = x.shape
    T = B * S
    nt = T // TM

    gate_logits = jnp.einsum('bsd,de->bse', x, Wg)
    weights, selected = jax.lax.top_k(gate_logits, TOP_K)
    weights = jax.nn.softmax(weights, axis=2)

    sel_f = selected.reshape(T, TOP_K)
    wts_f = weights.reshape(T, TOP_K)
    e_ids = jnp.arange(NUM_EXPERTS, dtype=sel_f.dtype)
    wmap = jnp.einsum('tk,etk->et', wts_f,
                      (sel_f[None, :, :] == e_ids[:, None, None]).astype(wts_f.dtype))
    wvec = wmap.reshape(NUM_EXPERTS * nt, 1, TM)

    xf = x.reshape(T, D)

    out = pl.pallas_call(
        _moe_body,
        grid=(nt, NUM_EXPERTS),
        in_specs=[
            pl.BlockSpec((1, 1, TM), lambda i, e: (e * nt + i, 0, 0)),
            pl.BlockSpec((TM, D_MODEL), lambda i, e: (i, 0)),
            pl.BlockSpec((1, D_MODEL, D_FF), lambda i, e: (e, 0, 0)),
            pl.BlockSpec((1, D_MODEL, D_FF), lambda i, e: (e, 0, 0)),
            pl.BlockSpec((1, D_FF, D_MODEL), lambda i, e: (e, 0, 0)),
        ],
        out_specs=pl.BlockSpec((TM, D_MODEL), lambda i, e: (i, 0)),
        out_shape=jax.ShapeDtypeStruct((T, D_MODEL), jnp.float32),
        compiler_params=pltpu.CompilerParams(
            vmem_limit_bytes=100 * 1024 * 1024),
    )(wvec, xf, w1, w3, w2)

    return out.reshape(B, S, D)


# speedup vs baseline: 1.6335x; 1.4982x over previous
import functools

import jax
import jax.numpy as jnp
from jax.experimental import pallas as pl
from jax.experimental.pallas import tpu as pltpu

NUM_EXPERTS = 8
TOP_K = 2
D_MODEL = 1024
D_FF = 2048

TM = 512


def _group_body(sched_ref, x_ref, gs_ref, w1_ref, w3_ref, w2_ref, y_ref):
    g = pl.program_id(0)
    lo = sched_ref[2, g]
    hi = sched_ref[3, g]
    first = sched_ref[4, g]
    t = sched_ref[0, g]

    @pl.when(hi > lo)
    def _compute():
        xb = x_ref[...]
        h1 = jnp.dot(xb, w1_ref[0], preferred_element_type=jnp.float32)
        h3 = jnp.dot(xb, w3_ref[0], preferred_element_type=jnp.float32)
        h = (h1 * jax.nn.sigmoid(h1)) * h3
        y = jnp.dot(h, w2_ref[0], preferred_element_type=jnp.float32)
        y = y * gs_ref[0, 0, :][:, None]
        rows = t * TM + jax.lax.broadcasted_iota(jnp.int32, (TM, 1), 0)
        mask = (rows >= lo) & (rows < hi)
        contrib = jnp.where(mask, y, 0.0)

        @pl.when(first == 1)
        def _init():
            y_ref[...] = contrib

        @pl.when(first == 0)
        def _acc():
            y_ref[...] += contrib


def kernel(x, Wg, w1, w3, w2):
    B, S, D = x.shape
    T = B * S
    A = T * TOP_K
    nt = A // TM
    G = nt + NUM_EXPERTS - 1

    gate_logits = jnp.einsum('bsd,de->bse', x, Wg)
    weights, selected = jax.lax.top_k(gate_logits, TOP_K)
    weights = jax.nn.softmax(weights, axis=2)

    e_flat = selected.reshape(A).astype(jnp.int32)
    g_flat = weights.reshape(A)

    sort_idx = jnp.argsort(e_flat, stable=True)
    tok_sorted = sort_idx // TOP_K
    gate_sorted = g_flat[sort_idx]
    inv = jnp.zeros((A,), jnp.int32).at[sort_idx].set(
        jnp.arange(A, dtype=jnp.int32))

    sizes = jnp.bincount(e_flat, length=NUM_EXPERTS).astype(jnp.int32)
    ends = jnp.cumsum(sizes)
    starts = ends - sizes

    t_start = starts // TM
    t_last = jnp.maximum(ends - 1, 0) // TM
    touched = jnp.where(sizes > 0, t_last - t_start + 1, 0)
    pair_end = jnp.cumsum(touched)
    pair_start = pair_end - touched

    gidx = jnp.arange(G, dtype=jnp.int32)
    e_of_g = jnp.searchsorted(pair_end, gidx, side='right').astype(jnp.int32)
    valid = e_of_g < NUM_EXPERTS
    e_cl = jnp.minimum(e_of_g, NUM_EXPERTS - 1)
    last_e = jnp.searchsorted(pair_end, pair_end[-1] - 1,
                              side='right').astype(jnp.int32)
    e_g = jnp.where(valid, e_cl, last_e)
    t_g = jnp.where(valid, t_start[e_cl] + (gidx - pair_start[e_cl]), nt - 1)
    lo_g = jnp.where(valid, jnp.maximum(starts[e_g], t_g * TM), 0)
    hi_g = jnp.where(valid, jnp.minimum(ends[e_g], (t_g + 1) * TM), 0)
    first_g = jnp.concatenate([
        jnp.ones((1,), jnp.int32),
        (t_g[1:] != t_g[:-1]).astype(jnp.int32),
    ])
    sched = jnp.stack([t_g, e_g, lo_g, hi_g, first_g])

    xf = x.reshape(T, D)
    x_sorted = jnp.take(xf, tok_sorted, axis=0)
    gs3 = gate_sorted.reshape(nt, 1, TM)

    grid_spec = pltpu.PrefetchScalarGridSpec(
        num_scalar_prefetch=1,
        grid=(G,),
        in_specs=[
            pl.BlockSpec((TM, D_MODEL), lambda g, s: (s[0, g], 0)),
            pl.BlockSpec((1, 1, TM), lambda g, s: (s[0, g], 0, 0)),
            pl.BlockSpec((1, D_MODEL, D_FF), lambda g, s: (s[1, g], 0, 0)),
            pl.BlockSpec((1, D_MODEL, D_FF), lambda g, s: (s[1, g], 0, 0)),
            pl.BlockSpec((1, D_FF, D_MODEL), lambda g, s: (s[1, g], 0, 0)),
        ],
        out_specs=pl.BlockSpec((TM, D_MODEL), lambda g, s: (s[0, g], 0)),
    )
    y_sorted = pl.pallas_call(
        _group_body,
        grid_spec=grid_spec,
        out_shape=jax.ShapeDtypeStruct((A, D_MODEL), jnp.float32),
        compiler_params=pltpu.CompilerParams(
            vmem_limit_bytes=100 * 1024 * 1024),
    )(sched, x_sorted, gs3, w1, w3, w2)

    inv2 = inv.reshape(T, TOP_K)
    out = jnp.take(y_sorted, inv2[:, 0], axis=0) + \
        jnp.take(y_sorted, inv2[:, 1], axis=0)
    return out.reshape(B, S, D)
